# baseline (speedup 1.0000x reference)
import functools
import math

import jax
import jax.numpy as jnp
from jax import lax
from jax.experimental import pallas as pl
from jax.experimental.pallas import tpu as pltpu
from jax.experimental.pallas import tpu_sc as plsc

D_MODEL = 128
SCALE = math.sqrt(float(D_MODEL))
LANES = 16

NUM_CORES = 2
NUM_SUBCORES = 16
NW = NUM_CORES * NUM_SUBCORES

N_SEQ = 4096
SEQ_LEN = 50
SEQ_PER_W = N_SEQ // NW
N_CHUNKS = SEQ_LEN
CHUNK = SEQ_PER_W
NBUF = 6
LAG = 3

_mesh = plsc.VectorSubcoreMesh(core_axis_name="c", subcore_axis_name="s")


def _scale_buf(buf):
    @plsc.parallel_loop(0, CHUNK, step=1, unroll=2)
    def _row(r):
        for k in range(D_MODEL // LANES):
            sl = (r, pl.ds(k * LANES, LANES))
            buf[sl] = buf[sl] * SCALE


@functools.partial(
    pl.kernel,
    out_type=jax.ShapeDtypeStruct((SEQ_LEN, N_SEQ, D_MODEL), jnp.float32),
    mesh=_mesh,
    compiler_params=pltpu.CompilerParams(use_tc_tiling_on_sc=True),
    scratch_types=[
        pltpu.VMEM((N_CHUNKS, CHUNK), jnp.int32),
        [pltpu.VMEM((CHUNK, D_MODEL), jnp.float32)] * NBUF,
        [pltpu.SemaphoreType.DMA] * NBUF,
        [pltpu.SemaphoreType.DMA] * NBUF,
    ],
)
def _emb_lookup(xt_hbm, lut_hbm, out_hbm, idx_v, bufs, gsems, osems):
    wid = lax.axis_index("s") * NUM_CORES + lax.axis_index("c")
    s0 = wid * SEQ_PER_W

    def gather_start(j, bi):
        pltpu.async_copy(lut_hbm.at[idx_v.at[j]], bufs[bi], gsems[bi])

    def gather_wait(j, bi):
        pltpu.make_async_copy(lut_hbm.at[idx_v.at[j]], bufs[bi], gsems[bi]).wait()

    def out_start(j, bi):
        pltpu.async_copy(bufs[bi], out_hbm.at[j, pl.ds(s0, CHUNK)], osems[bi])

    def out_wait(j, bi):
        pltpu.make_async_copy(
            bufs[bi], out_hbm.at[j, pl.ds(s0, CHUNK)], osems[bi]
        ).wait()

    pltpu.sync_copy(xt_hbm.at[:, wid], idx_v)

    for j in range(LAG):
        gather_start(j, j)

    def chunk_step(j, bi, wait_out, ahead):
        gather_wait(j, bi)
        if wait_out:
            out_wait(j - LAG, (bi - LAG) % NBUF)
        if ahead:
            gather_start(j + LAG, (bi + LAG) % NBUF)
        _scale_buf(bufs[bi])
        out_start(j, bi)

    for j in range(LAG):
        chunk_step(j, j % NBUF, wait_out=False, ahead=True)
    for j in range(LAG, 2 * LAG):
        chunk_step(j, j % NBUF, wait_out=True, ahead=True)

    n_main = (N_CHUNKS - 3 * LAG) // NBUF * NBUF
    main_lo = 2 * LAG

    def ring_body(g, carry):
        for b in range(NBUF):
            j = main_lo + NBUF * g + b
            chunk_step(j, b, wait_out=True, ahead=True)
        return carry

    lax.fori_loop(0, n_main // NBUF, ring_body, 0, unroll=False)

    for j in range(main_lo + n_main, N_CHUNKS - LAG):
        chunk_step(j, j % NBUF, wait_out=True, ahead=True)
    for j in range(N_CHUNKS - LAG, N_CHUNKS):
        chunk_step(j, j % NBUF, wait_out=True, ahead=False)
    for j in range(N_CHUNKS - LAG, N_CHUNKS):
        out_wait(j, j % NBUF)


def kernel(x, lut):
    xt = jnp.transpose(x.astype(jnp.int32), (1, 0)).reshape(
        SEQ_LEN, NW, SEQ_PER_W
    )
    out_tmajor = _emb_lookup(xt, lut)
    return jnp.transpose(out_tmajor, (1, 0, 2))


# speedup vs baseline: 1.0022x; 1.0022x over previous
import functools
import math

import jax
import jax.numpy as jnp
from jax import lax
from jax.experimental import pallas as pl
from jax.experimental.pallas import tpu as pltpu
from jax.experimental.pallas import tpu_sc as plsc

D_MODEL = 128
SCALE = math.sqrt(float(D_MODEL))
LANES = 16

NUM_CORES = 2
NUM_SUBCORES = 16
NW = NUM_CORES * NUM_SUBCORES

N_SEQ = 4096
SEQ_LEN = 50
SEQ_PER_W = N_SEQ // NW
N_CHUNKS = SEQ_LEN
CHUNK = SEQ_PER_W
NBUF = 4
LAG = 2

_mesh = plsc.VectorSubcoreMesh(core_axis_name="c", subcore_axis_name="s")


def _scale_buf(buf):
    @plsc.parallel_loop(0, CHUNK, step=1, unroll=2)
    def _row(r):
        for k in range(D_MODEL // LANES):
            sl = (r, pl.ds(k * LANES, LANES))
            buf[sl] = buf[sl] * SCALE


@functools.partial(
    pl.kernel,
    out_type=jax.ShapeDtypeStruct((SEQ_LEN, N_SEQ, D_MODEL), jnp.float32),
    mesh=_mesh,
    compiler_params=pltpu.CompilerParams(
        use_tc_tiling_on_sc=True,
        disable_bounds_checks=True,
        disable_semaphore_checks=True,
        skip_device_barrier=True,
    ),
    scratch_types=[
        pltpu.VMEM((N_CHUNKS, CHUNK), jnp.int32),
        [pltpu.VMEM((CHUNK, D_MODEL), jnp.float32)] * NBUF,
        [pltpu.SemaphoreType.DMA] * NBUF,
        [pltpu.SemaphoreType.DMA] * NBUF,
    ],
)
def _emb_lookup(xt_hbm, lut_hbm, out_hbm, idx_v, bufs, gsems, osems):
    wid = lax.axis_index("s") * NUM_CORES + lax.axis_index("c")
    s0 = wid * SEQ_PER_W

    def gather_start(j, bi):
        pltpu.async_copy(lut_hbm.at[idx_v.at[j]], bufs[bi], gsems[bi])

    def gather_wait(j, bi):
        pltpu.make_async_copy(lut_hbm.at[idx_v.at[j]], bufs[bi], gsems[bi]).wait()

    def out_start(j, bi):
        pltpu.async_copy(bufs[bi], out_hbm.at[j, pl.ds(s0, CHUNK)], osems[bi])

    def out_wait(j, bi):
        pltpu.make_async_copy(
            bufs[bi], out_hbm.at[j, pl.ds(s0, CHUNK)], osems[bi]
        ).wait()

    pltpu.sync_copy(xt_hbm.at[:, wid], idx_v)

    for j in range(LAG):
        gather_start(j, j)

    def chunk_step(j, bi, wait_out, ahead):
        gather_wait(j, bi)
        if wait_out:
            out_wait(j - LAG, (bi - LAG) % NBUF)
        if ahead:
            gather_start(j + LAG, (bi + LAG) % NBUF)
        _scale_buf(bufs[bi])
        out_start(j, bi)

    for j in range(LAG):
        chunk_step(j, j % NBUF, wait_out=False, ahead=True)
    for j in range(LAG, 2 * LAG):
        chunk_step(j, j % NBUF, wait_out=True, ahead=True)

    n_main = (N_CHUNKS - 3 * LAG) // NBUF * NBUF
    main_lo = 2 * LAG

    def ring_body(g, carry):
        for b in range(NBUF):
            j = main_lo + NBUF * g + b
            chunk_step(j, b, wait_out=True, ahead=True)
        return carry

    lax.fori_loop(0, n_main // NBUF, ring_body, 0, unroll=False)

    for j in range(main_lo + n_main, N_CHUNKS - LAG):
        chunk_step(j, j % NBUF, wait_out=True, ahead=True)
    for j in range(N_CHUNKS - LAG, N_CHUNKS):
        chunk_step(j, j % NBUF, wait_out=True, ahead=False)
    for j in range(N_CHUNKS - LAG, N_CHUNKS):
        out_wait(j, j % NBUF)


def kernel(x, lut):
    xt = jnp.transpose(x.astype(jnp.int32), (1, 0)).reshape(
        SEQ_LEN, NW, SEQ_PER_W
    )
    out_tmajor = _emb_lookup(xt, lut)
    return jnp.transpose(out_tmajor, (1, 0, 2))


# speedup vs baseline: 1.0041x; 1.0019x over previous
import functools
import math

import jax
import jax.numpy as jnp
from jax import lax
from jax.experimental import pallas as pl
from jax.experimental.pallas import tpu as pltpu
from jax.experimental.pallas import tpu_sc as plsc

D_MODEL = 128
SCALE = math.sqrt(float(D_MODEL))
LANES = 16

NUM_CORES = 2
NUM_SUBCORES = 16
NW = NUM_CORES * NUM_SUBCORES

N_SEQ = 4096
SEQ_LEN = 50
SEQ_PER_W = N_SEQ // NW
N_CHUNKS = SEQ_LEN
CHUNK = SEQ_PER_W
NBUF = 4
LAG = 2

_mesh = plsc.VectorSubcoreMesh(core_axis_name="c", subcore_axis_name="s")


def _scale_buf(buf):
    @plsc.parallel_loop(0, CHUNK, step=1, unroll=2)
    def _row(r):
        for k in range(D_MODEL // LANES):
            sl = (r, pl.ds(k * LANES, LANES))
            buf[sl] = buf[sl] * SCALE


@functools.partial(
    pl.kernel,
    out_type=jax.ShapeDtypeStruct((SEQ_LEN, N_SEQ, D_MODEL), jnp.float32),
    mesh=_mesh,
    compiler_params=pltpu.CompilerParams(use_tc_tiling_on_sc=True),
    scratch_types=[
        pltpu.VMEM((N_CHUNKS, CHUNK), jnp.int32),
        [pltpu.VMEM((CHUNK, D_MODEL), jnp.float32)] * NBUF,
        [pltpu.SemaphoreType.DMA] * NBUF,
        [pltpu.SemaphoreType.DMA] * NBUF,
    ],
)
def _emb_lookup(xt_hbm, lut_hbm, out_hbm, idx_v, bufs, gsems, osems):
    wid = lax.axis_index("s") * NUM_CORES + lax.axis_index("c")
    s0 = wid * SEQ_PER_W

    def gather_start(j, bi):
        pltpu.async_copy(lut_hbm.at[idx_v.at[j]], bufs[bi], gsems[bi])

    def gather_wait(j, bi):
        pltpu.make_async_copy(lut_hbm.at[idx_v.at[j]], bufs[bi], gsems[bi]).wait()

    def out_start(j, bi):
        pltpu.async_copy(bufs[bi], out_hbm.at[j, pl.ds(s0, CHUNK)], osems[bi])

    def out_wait(j, bi):
        pltpu.make_async_copy(
            bufs[bi], out_hbm.at[j, pl.ds(s0, CHUNK)], osems[bi]
        ).wait()

    pltpu.sync_copy(xt_hbm.at[:, wid], idx_v)

    for j in range(LAG):
        gather_start(j, j)

    def chunk_step(j, bi, wait_out, ahead):
        gather_wait(j, bi)
        if wait_out:
            out_wait(j - LAG, (bi - LAG) % NBUF)
        if ahead:
            gather_start(j + LAG, (bi + LAG) % NBUF)
        _scale_buf(bufs[bi])
        out_start(j, bi)

    for j in range(LAG):
        chunk_step(j, j % NBUF, wait_out=False, ahead=True)
    for j in range(LAG, 2 * LAG):
        chunk_step(j, j % NBUF, wait_out=True, ahead=True)

    n_main = (N_CHUNKS - 3 * LAG) // NBUF * NBUF
    main_lo = 2 * LAG

    def ring_body(g, carry):
        for b in range(NBUF):
            j = main_lo + NBUF * g + b
            chunk_step(j, b, wait_out=True, ahead=True)
        return carry

    lax.fori_loop(0, n_main // NBUF, ring_body, 0, unroll=False)

    for j in range(main_lo + n_main, N_CHUNKS - LAG):
        chunk_step(j, j % NBUF, wait_out=True, ahead=True)
    for j in range(N_CHUNKS - LAG, N_CHUNKS):
        chunk_step(j, j % NBUF, wait_out=True, ahead=False)
    for j in range(N_CHUNKS - LAG, N_CHUNKS):
        out_wait(j, j % NBUF)


def kernel(x, lut):
    xt = jnp.transpose(x.astype(jnp.int32), (1, 0)).reshape(
        SEQ_LEN, NW, SEQ_PER_W
    )
    out_tmajor = _emb_lookup(xt, lut)
    return jnp.transpose(out_tmajor, (1, 0, 2))


# speedup vs baseline: 1.0054x; 1.0013x over previous
import functools
import math

import jax
import jax.numpy as jnp
from jax import lax
from jax.experimental import pallas as pl
from jax.experimental.pallas import tpu as pltpu
from jax.experimental.pallas import tpu_sc as plsc

D_MODEL = 128
SCALE = math.sqrt(float(D_MODEL))
LANES = 16

NUM_CORES = 2
NUM_SUBCORES = 16
NW = NUM_CORES * NUM_SUBCORES

N_SEQ = 4096
SEQ_LEN = 50
SEQ_PER_W = N_SEQ // NW
N_CHUNKS = SEQ_LEN
CHUNK = SEQ_PER_W
NBUF = 4
LAG = 2

_mesh = plsc.VectorSubcoreMesh(core_axis_name="c", subcore_axis_name="s")


def _scale_buf(buf):
    @plsc.parallel_loop(0, CHUNK, step=1, unroll=2)
    def _row(r):
        for k in range(D_MODEL // LANES):
            sl = (r, pl.ds(k * LANES, LANES))
            buf[sl] = buf[sl] * SCALE


@functools.partial(
    pl.kernel,
    out_type=jax.ShapeDtypeStruct((SEQ_LEN, N_SEQ, D_MODEL), jnp.float32),
    mesh=_mesh,
    compiler_params=pltpu.CompilerParams(use_tc_tiling_on_sc=True),
    scratch_types=[
        pltpu.VMEM((N_CHUNKS, CHUNK), jnp.int32),
        [pltpu.VMEM((CHUNK, D_MODEL), jnp.float32)] * NBUF,
        [pltpu.SemaphoreType.DMA] * NBUF,
        [pltpu.SemaphoreType.DMA] * NBUF,
    ],
)
def _emb_lookup(xt_hbm, lut_hbm, out_hbm, idx_v, bufs, gsems, osems):
    wid = lax.axis_index("s") * NUM_CORES + lax.axis_index("c")
    s0 = wid * SEQ_PER_W

    HALF = CHUNK // 2

    def gather_start(j, bi):
        for h in range(2):
            pltpu.async_copy(
                lut_hbm.at[idx_v.at[j, pl.ds(h * HALF, HALF)]],
                bufs[bi].at[pl.ds(h * HALF, HALF)],
                gsems[bi],
            )

    def gather_wait(j, bi):
        for h in range(2):
            pltpu.make_async_copy(
                lut_hbm.at[idx_v.at[j, pl.ds(h * HALF, HALF)]],
                bufs[bi].at[pl.ds(h * HALF, HALF)],
                gsems[bi],
            ).wait()

    def out_start(j, bi):
        pltpu.async_copy(bufs[bi], out_hbm.at[j, pl.ds(s0, CHUNK)], osems[bi])

    def out_wait(j, bi):
        pltpu.make_async_copy(
            bufs[bi], out_hbm.at[j, pl.ds(s0, CHUNK)], osems[bi]
        ).wait()

    pltpu.sync_copy(xt_hbm.at[:, wid], idx_v)

    for j in range(LAG):
        gather_start(j, j)

    def chunk_step(j, bi, wait_out, ahead):
        gather_wait(j, bi)
        if wait_out:
            out_wait(j - LAG, (bi - LAG) % NBUF)
        if ahead:
            gather_start(j + LAG, (bi + LAG) % NBUF)
        _scale_buf(bufs[bi])
        out_start(j, bi)

    for j in range(LAG):
        chunk_step(j, j % NBUF, wait_out=False, ahead=True)
    for j in range(LAG, 2 * LAG):
        chunk_step(j, j % NBUF, wait_out=True, ahead=True)

    n_main = (N_CHUNKS - 3 * LAG) // NBUF * NBUF
    main_lo = 2 * LAG

    def ring_body(g, carry):
        for b in range(NBUF):
            j = main_lo + NBUF * g + b
            chunk_step(j, b, wait_out=True, ahead=True)
        return carry

    lax.fori_loop(0, n_main // NBUF, ring_body, 0, unroll=False)

    for j in range(main_lo + n_main, N_CHUNKS - LAG):
        chunk_step(j, j % NBUF, wait_out=True, ahead=True)
    for j in range(N_CHUNKS - LAG, N_CHUNKS):
        chunk_step(j, j % NBUF, wait_out=True, ahead=False)
    for j in range(N_CHUNKS - LAG, N_CHUNKS):
        out_wait(j, j % NBUF)


def kernel(x, lut):
    xt = jnp.transpose(x.astype(jnp.int32), (1, 0)).reshape(
        SEQ_LEN, NW, SEQ_PER_W
    )
    out_tmajor = _emb_lookup(xt, lut)
    return jnp.transpose(out_tmajor, (1, 0, 2))
